# baseline (speedup 1.0000x reference)
import functools

import jax
import jax.numpy as jnp
from jax import lax
from jax.experimental import pallas as pl
from jax.experimental.pallas import tpu as pltpu
from jax.experimental.pallas import tpu_sc as plsc

_INFO = plsc.get_sparse_core_info()
_NC, _NS = _INFO.num_cores, _INFO.num_subcores
_NW = _NC * _NS

_BATCH = 16
_SEQ = 1024
_B = _BATCH * _SEQ
_D = 64
_DW = 2 * _D
_L = 16
_BPW = _B // _NW
_CHUNK = 64
_NCHUNK = _BPW // _CHUNK
_WPB = _SEQ // _BPW

_V = 512
_RPS = _V // _NS

_mesh = plsc.VectorSubcoreMesh(core_axis_name="c", subcore_axis_name="s")


@functools.partial(
    pl.kernel,
    mesh=_mesh,
    out_type=jax.ShapeDtypeStruct((_BATCH, _SEQ, _D), jnp.float32),
    scratch_types=[
        pltpu.VMEM((_BPW,), jnp.int32),
        pltpu.VMEM((2, _CHUNK, _DW), jnp.float32),
        pltpu.VMEM((_BPW, _D), jnp.float32),
        pltpu.VMEM_SHARED((_V, _DW), jnp.float32),
        pltpu.SemaphoreType.DMA((_NCHUNK,)),
        pltpu.SemaphoreType.DMA((_NCHUNK,)),
    ],
)
def _gather_kernel(table_hbm, idx_hbm, out_hbm, idx_v, rows_v, outc_v, table_s, gsem, wsem):
    cid = lax.axis_index("c")
    sid = lax.axis_index("s")
    wid = sid * _NC + cid
    row = wid // _WPB
    off = (wid % _WPB) * _BPW
    pltpu.sync_copy(
        table_hbm.at[pl.ds(sid * _RPS, _RPS)],
        table_s.at[pl.ds(sid * _RPS, _RPS)],
    )
    pltpu.sync_copy(idx_hbm.at[row, pl.ds(off, _BPW)], idx_v)
    plsc.subcore_barrier()
    for j in range(2):
        pltpu.async_copy(
            table_s.at[idx_v.at[pl.ds(j * _CHUNK, _CHUNK)]],
            rows_v.at[j],
            gsem.at[j],
        )
    for j in range(_NCHUNK):
        slot = j % 2
        pltpu.make_async_copy(
            table_s.at[idx_v.at[pl.ds(j * _CHUNK, _CHUNK)]],
            rows_v.at[slot],
            gsem.at[j],
        ).wait()

        def _compact(i, carry, j=j, slot=slot):
            r = i * 4
            for u in range(4):
                for c in range(_D // _L):
                    outc_v[j * _CHUNK + r + u, pl.ds(c * _L, _L)] = (
                        rows_v[slot, r + u, pl.ds(c * _L, _L)]
                    )
            return carry

        lax.fori_loop(0, _CHUNK // 4, _compact, 0)
        if j + 2 < _NCHUNK:
            pltpu.async_copy(
                table_s.at[idx_v.at[pl.ds((j + 2) * _CHUNK, _CHUNK)]],
                rows_v.at[slot],
                gsem.at[j + 2],
            )
        pltpu.async_copy(
            outc_v.at[pl.ds(j * _CHUNK, _CHUNK)],
            out_hbm.at[row, pl.ds(off + j * _CHUNK, _CHUNK)],
            wsem.at[j],
        )
    for j in range(_NCHUNK):
        pltpu.make_async_copy(
            outc_v.at[pl.ds(j * _CHUNK, _CHUNK)],
            out_hbm.at[row, pl.ds(off + j * _CHUNK, _CHUNK)],
            wsem.at[j],
        ).wait()


def kernel(x, codebook):
    wide = jnp.concatenate([codebook, codebook], axis=1)
    return _gather_kernel(wide, x.astype(jnp.int32))


# speedup vs baseline: 1.0295x; 1.0295x over previous
import functools

import jax
import jax.numpy as jnp
from jax import lax
from jax.experimental import pallas as pl
from jax.experimental.pallas import tpu as pltpu
from jax.experimental.pallas import tpu_sc as plsc

_INFO = plsc.get_sparse_core_info()
_NC, _NS = _INFO.num_cores, _INFO.num_subcores
_NW = _NC * _NS

_BATCH = 16
_SEQ = 1024
_B = _BATCH * _SEQ
_D = 64
_DW = 2 * _D
_L = 16
_BPW = _B // _NW
_CHUNK = 128
_NCHUNK = _BPW // _CHUNK
_WPB = _SEQ // _BPW

_V = 512
_RPS = _V // _NS

_mesh = plsc.VectorSubcoreMesh(core_axis_name="c", subcore_axis_name="s")


@functools.partial(
    pl.kernel,
    mesh=_mesh,
    out_type=jax.ShapeDtypeStruct((_BATCH, _SEQ, _D), jnp.float32),
    scratch_types=[
        pltpu.VMEM((_BPW,), jnp.int32),
        pltpu.VMEM((2, _CHUNK, _DW), jnp.float32),
        pltpu.VMEM((_BPW, _D), jnp.float32),
        pltpu.VMEM_SHARED((_V, _DW), jnp.float32),
        pltpu.SemaphoreType.DMA((_NCHUNK,)),
        pltpu.SemaphoreType.DMA((_NCHUNK,)),
    ],
)
def _gather_kernel(table_hbm, idx_hbm, out_hbm, idx_v, rows_v, outc_v, table_s, gsem, wsem):
    cid = lax.axis_index("c")
    sid = lax.axis_index("s")
    wid = sid * _NC + cid
    row = wid // _WPB
    off = (wid % _WPB) * _BPW
    pltpu.async_copy(
        table_hbm.at[pl.ds(sid * _RPS, _RPS)],
        table_s.at[pl.ds(sid * _RPS, _RPS)],
        wsem.at[0],
    )
    pltpu.sync_copy(idx_hbm.at[row, pl.ds(off, _BPW)], idx_v)
    pltpu.make_async_copy(
        table_hbm.at[pl.ds(sid * _RPS, _RPS)],
        table_s.at[pl.ds(sid * _RPS, _RPS)],
        wsem.at[0],
    ).wait()
    plsc.subcore_barrier()
    for j in range(2):
        pltpu.async_copy(
            table_s.at[idx_v.at[pl.ds(j * _CHUNK, _CHUNK)]],
            rows_v.at[j],
            gsem.at[j],
        )
    for j in range(_NCHUNK):
        slot = j % 2
        pltpu.make_async_copy(
            table_s.at[idx_v.at[pl.ds(j * _CHUNK, _CHUNK)]],
            rows_v.at[slot],
            gsem.at[j],
        ).wait()

        def _compact(i, carry, j=j, slot=slot):
            r = i * 4
            for u in range(4):
                for c in range(_D // _L):
                    outc_v[j * _CHUNK + r + u, pl.ds(c * _L, _L)] = (
                        rows_v[slot, r + u, pl.ds(c * _L, _L)]
                    )
            return carry

        lax.fori_loop(0, _CHUNK // 4, _compact, 0)
        if j + 2 < _NCHUNK:
            pltpu.async_copy(
                table_s.at[idx_v.at[pl.ds((j + 2) * _CHUNK, _CHUNK)]],
                rows_v.at[slot],
                gsem.at[j + 2],
            )
        pltpu.async_copy(
            outc_v.at[pl.ds(j * _CHUNK, _CHUNK)],
            out_hbm.at[row, pl.ds(off + j * _CHUNK, _CHUNK)],
            wsem.at[j],
        )
    for j in range(_NCHUNK):
        pltpu.make_async_copy(
            outc_v.at[pl.ds(j * _CHUNK, _CHUNK)],
            out_hbm.at[row, pl.ds(off + j * _CHUNK, _CHUNK)],
            wsem.at[j],
        ).wait()


def kernel(x, codebook):
    wide = jnp.concatenate([codebook, codebook], axis=1)
    return _gather_kernel(wide, x.astype(jnp.int32))
